# baseline (speedup 1.0000x reference)
import math

import jax
import jax.numpy as jnp
from jax.experimental import pallas as pl
from jax.experimental.pallas import tpu as pltpu

_B, _L, _H, _D = 32, 256, 16, 32
_C, _K = 256, 8
_ICH = _L * _D
_CK = _C * _K
_BH = _B * _H
_NT = 30
_UP = 32
_TL = 8
_KB = _TL * _D
_NKB = _ICH // _KB
_G = 32
_SCALE = 1.0 / math.sqrt(_D)
_PH = jax.lax.Precision.HIGHEST
_PM = jax.lax.Precision.HIGH


def _proj_kernel(aq_ref, ak_ref, wq_ref, wk_ref, bq_ref, bk_ref, oq_ref, ok_ref):
    kb = pl.program_id(0)

    @pl.when(kb == 0)
    def _init():
        oq_ref[...] = jnp.zeros_like(oq_ref)
        ok_ref[...] = jnp.zeros_like(ok_ref)

    aq = jnp.concatenate(
        [aq_ref[:, l, :, :].reshape(_BH, _D).astype(jnp.bfloat16)
         for l in range(_TL)], axis=1)
    ak = jnp.concatenate(
        [ak_ref[:, l, :, :].reshape(_BH, _D).astype(jnp.bfloat16)
         for l in range(_TL)], axis=1)
    dn = (((1,), (1,)), ((), ()))
    for kk in range(_K):
        wq = wq_ref[:, kk, :].astype(jnp.bfloat16)
        wk = wk_ref[:, kk, :].astype(jnp.bfloat16)
        oq_ref[:, kk, :] += jax.lax.dot_general(
            aq, wq, dn, preferred_element_type=jnp.float32)
        ok_ref[:, kk, :] += jax.lax.dot_general(
            ak, wk, dn, preferred_element_type=jnp.float32)

    @pl.when(kb == _NKB - 1)
    def _bias():
        oq_ref[...] += bq_ref[...][None, :, :]
        ok_ref[...] += bk_ref[...][None, :, :]


def _attn_kernel(qw_ref, kw_ref, v_ref, cntT_ref, o_ref):
    cntT = cntT_ref[...]
    masknegT = jnp.where(cntT > 0.0, 0.0, -jnp.inf)
    rowi = jax.lax.broadcasted_iota(jnp.int32, (_C, _C), 0)
    coli = jax.lax.broadcasted_iota(jnp.int32, (_C, _C), 1)
    tril = (coli <= rowi).astype(jnp.float32)
    l_iota = jax.lax.broadcasted_iota(jnp.int32, (_G, _C), 1)
    m_colf = jax.lax.broadcasted_iota(jnp.int32, (_C, 1), 0).astype(jnp.float32)
    u_row = jax.lax.broadcasted_iota(jnp.int32, (1, _UP), 1)

    ms = []
    for g in range(_G):
        qw = qw_ref[g]
        kw = kw_ref[g]
        st = jax.lax.dot_general(
            kw, qw, (((0,), (0,)), ((), ())),
            preferred_element_type=jnp.float32, precision=_PH)
        colmax = jnp.max(st + masknegT, axis=0, keepdims=True)
        colsum = jnp.sum(st * cntT, axis=0, keepdims=True)
        ms.append(colmax - colsum * (1.0 / _C))

    m_all = jnp.concatenate(ms, axis=0)

    def body(u, carry):
        mv, order = carry
        mx = jnp.max(mv, axis=1, keepdims=True)
        idx = jnp.min(jnp.where(mv == mx, l_iota, _C), axis=1, keepdims=True)
        pick = l_iota == idx
        order = jnp.where(pick, u + 1, order)
        mv = jnp.where(pick, -jnp.inf, mv)
        return mv, order

    _, order = jax.lax.fori_loop(
        0, _NT, body, (m_all, jnp.zeros((_G, _C), jnp.int32)))
    ordt = jnp.transpose(order, (1, 0))

    for g in range(_G):
        v = v_ref[(g // _H) * _L:(g // _H + 1) * _L,
                  (g % _H) * _D:(g % _H + 1) * _D]
        uoht = (ordt[:, g:g + 1] == u_row + 1).astype(jnp.float32)
        qred = jax.lax.dot_general(
            qw_ref[g], uoht, (((1,), (0,)), ((), ())),
            preferred_element_type=jnp.float32, precision=jax.lax.Precision.DEFAULT)
        scores = jax.lax.dot_general(
            kw_ref[g], qred, (((0,), (0,)), ((), ())),
            preferred_element_type=jnp.float32, precision=jax.lax.Precision.DEFAULT) * _SCALE
        idxrow = jnp.sum(uoht * m_colf, axis=0, keepdims=True)
        scores = jnp.where(m_colf > idxrow, -jnp.inf, scores)
        smax = jnp.max(scores, axis=0, keepdims=True)
        e = jnp.exp(scores - smax)
        attn = e / jnp.sum(e, axis=0, keepdims=True)
        upd = jax.lax.dot_general(
            attn, v, (((0,), (0,)), ((), ())),
            preferred_element_type=jnp.float32, precision=jax.lax.Precision.DEFAULT)
        ctx = jax.lax.dot_general(
            tril, v, (((1,), (0,)), ((), ())),
            preferred_element_type=jnp.float32, precision=jax.lax.Precision.DEFAULT)
        scat = jax.lax.dot_general(
            uoht, upd, (((1,), (0,)), ((), ())),
            preferred_element_type=jnp.float32, precision=jax.lax.Precision.DEFAULT)
        sel = jnp.sum(uoht, axis=1, keepdims=True)
        o_ref[g] = ctx * (1.0 - sel) + scat


def kernel(queries, keys, values, attn_mask, Wq_w, Wq_b, Wk_w, Wk_b,
           Wv_w, Wv_b, index_sample):
    del attn_mask, Wv_w, Wv_b
    v2 = values.reshape(_B * _L, _H * _D)
    w3q = Wq_w.reshape(_C, _K, _ICH)
    w3k = Wk_w.reshape(_C, _K, _ICH)
    bq = jnp.transpose(Wq_b.reshape(_C, _K), (1, 0))
    bk = jnp.transpose(Wk_b.reshape(_C, _K), (1, 0))
    cntt = jnp.sum(
        (jnp.arange(_C)[:, None, None] == index_sample[None, :, :]),
        axis=2).astype(jnp.float32)

    qw, kw = pl.pallas_call(
        _proj_kernel,
        grid=(_NKB,),
        in_specs=[
            pl.BlockSpec((_B, _TL, _H, _D), lambda kb: (0, kb, 0, 0)),
            pl.BlockSpec((_B, _TL, _H, _D), lambda kb: (0, kb, 0, 0)),
            pl.BlockSpec((_C, _K, _KB), lambda kb: (0, 0, kb)),
            pl.BlockSpec((_C, _K, _KB), lambda kb: (0, 0, kb)),
            pl.BlockSpec((_K, _C), lambda kb: (0, 0)),
            pl.BlockSpec((_K, _C), lambda kb: (0, 0)),
        ],
        out_specs=[
            pl.BlockSpec((_BH, _K, _C), lambda kb: (0, 0, 0)),
            pl.BlockSpec((_BH, _K, _C), lambda kb: (0, 0, 0)),
        ],
        out_shape=[
            jax.ShapeDtypeStruct((_BH, _K, _C), jnp.float32),
            jax.ShapeDtypeStruct((_BH, _K, _C), jnp.float32),
        ],
        compiler_params=pltpu.CompilerParams(
            dimension_semantics=("arbitrary",)),
    )(queries, keys, w3q, w3k, bq, bk)

    ctx = pl.pallas_call(
        _attn_kernel,
        grid=(_BH // _G,),
        in_specs=[
            pl.BlockSpec((_G, _K, _C), lambda i: (i, 0, 0)),
            pl.BlockSpec((_G, _K, _C), lambda i: (i, 0, 0)),
            pl.BlockSpec((_G // _H * _L, _H * _D), lambda i: (i, 0)),
            pl.BlockSpec((_C, _C), lambda i: (0, 0)),
        ],
        out_specs=pl.BlockSpec((_G, _L, _D), lambda i: (i, 0, 0)),
        out_shape=jax.ShapeDtypeStruct((_BH, _L, _D), jnp.float32),
        compiler_params=pltpu.CompilerParams(
            dimension_semantics=("parallel",)),
    )(qw, kw, v2, cntt)

    return ctx.reshape(_B, _H, _L, _D)


# speedup vs baseline: 1.2151x; 1.2151x over previous
import math

import jax
import jax.numpy as jnp
from jax.experimental import pallas as pl
from jax.experimental.pallas import tpu as pltpu

_B, _L, _H, _D = 32, 256, 16, 32
_C, _K = 256, 8
_ICH = _L * _D
_CK = _C * _K
_BH = _B * _H
_NT = 30
_UP = 32
_TL = 8
_KB = _TL * _D
_NKB = _ICH // _KB
_G = 32
_SCALE = 1.0 / math.sqrt(_D)
_PH = jax.lax.Precision.HIGHEST
_PM = jax.lax.Precision.HIGH


def _proj_kernel(aq_ref, ak_ref, wq_ref, wk_ref, bq_ref, bk_ref, oq_ref, ok_ref):
    kb = pl.program_id(0)

    @pl.when(kb == 0)
    def _init():
        oq_ref[...] = jnp.zeros_like(oq_ref)
        ok_ref[...] = jnp.zeros_like(ok_ref)

    aq = jnp.concatenate(
        [aq_ref[:, l, :, :].reshape(_BH, _D) for l in range(_TL)], axis=1)
    ak = jnp.concatenate(
        [ak_ref[:, l, :, :].reshape(_BH, _D) for l in range(_TL)], axis=1)
    dn = (((1,), (1,)), ((), ()))
    for kk in range(_K):
        wq = wq_ref[:, kk, :]
        wk = wk_ref[:, kk, :]
        oq_ref[:, kk, :] += jax.lax.dot_general(
            aq, wq, dn, preferred_element_type=jnp.float32)
        ok_ref[:, kk, :] += jax.lax.dot_general(
            ak, wk, dn, preferred_element_type=jnp.float32)

    @pl.when(kb == _NKB - 1)
    def _bias():
        oq_ref[...] += bq_ref[...][None, :, :]
        ok_ref[...] += bk_ref[...][None, :, :]


def _attn_kernel(qw_ref, kw_ref, v_ref, cntT_ref, o_ref):
    cntT = cntT_ref[...]
    masknegT = jnp.where(cntT > 0.0, 0.0, -jnp.inf)
    rowi = jax.lax.broadcasted_iota(jnp.int32, (_C, _C), 0)
    coli = jax.lax.broadcasted_iota(jnp.int32, (_C, _C), 1)
    tril = (coli <= rowi).astype(jnp.float32)
    l_iota = jax.lax.broadcasted_iota(jnp.int32, (_G, _C), 1)
    m_colf = jax.lax.broadcasted_iota(jnp.int32, (_C, 1), 0).astype(jnp.float32)
    u_row = jax.lax.broadcasted_iota(jnp.int32, (1, _UP), 1)

    ms = []
    for g in range(_G):
        qw = qw_ref[g]
        kw = kw_ref[g]
        st = jax.lax.dot_general(
            kw, qw, (((0,), (0,)), ((), ())),
            preferred_element_type=jnp.float32, precision=_PH)
        colmax = jnp.max(st + masknegT, axis=0, keepdims=True)
        colsum = jnp.sum(st * cntT, axis=0, keepdims=True)
        ms.append(colmax - colsum * (1.0 / _C))

    m_all = jnp.concatenate(ms, axis=0)
    m_t = jnp.transpose(m_all, (1, 0))
    l_col = jax.lax.broadcasted_iota(jnp.int32, (_C, _G), 0)

    def body(u, carry):
        mv, order = carry
        mx = jnp.max(mv, axis=0, keepdims=True)
        idx = jnp.min(jnp.where(mv == mx, l_col, _C), axis=0, keepdims=True)
        pick = l_col == idx
        order = jnp.where(pick, u + 1, order)
        mv = jnp.where(pick, -jnp.inf, mv)
        return mv, order

    _, ordt = jax.lax.fori_loop(
        0, _NT, body, (m_t, jnp.zeros((_C, _G), jnp.int32)))

    for g in range(_G):
        v = v_ref[(g // _H) * _L:(g // _H + 1) * _L,
                  (g % _H) * _D:(g % _H + 1) * _D]
        uoht = (ordt[:, g:g + 1] == u_row + 1).astype(jnp.float32)
        qred = jax.lax.dot_general(
            qw_ref[g], uoht, (((1,), (0,)), ((), ())),
            preferred_element_type=jnp.float32, precision=jax.lax.Precision.DEFAULT)
        scores = jax.lax.dot_general(
            kw_ref[g], qred, (((0,), (0,)), ((), ())),
            preferred_element_type=jnp.float32, precision=jax.lax.Precision.DEFAULT) * _SCALE
        idxrow = jnp.sum(uoht * m_colf, axis=0, keepdims=True)
        scores = jnp.where(m_colf > idxrow, -jnp.inf, scores)
        smax = jnp.max(scores, axis=0, keepdims=True)
        e = jnp.exp(scores - smax)
        attn = e / jnp.sum(e, axis=0, keepdims=True)
        upd = jax.lax.dot_general(
            attn, v, (((0,), (0,)), ((), ())),
            preferred_element_type=jnp.float32, precision=jax.lax.Precision.DEFAULT)
        ctx = jax.lax.dot_general(
            tril, v, (((1,), (0,)), ((), ())),
            preferred_element_type=jnp.float32, precision=jax.lax.Precision.DEFAULT)
        scat = jax.lax.dot_general(
            uoht, upd, (((1,), (0,)), ((), ())),
            preferred_element_type=jnp.float32, precision=jax.lax.Precision.DEFAULT)
        sel = jnp.sum(uoht, axis=1, keepdims=True)
        o_ref[g] = ctx * (1.0 - sel) + scat


def kernel(queries, keys, values, attn_mask, Wq_w, Wq_b, Wk_w, Wk_b,
           Wv_w, Wv_b, index_sample):
    del attn_mask, Wv_w, Wv_b
    v2 = values.reshape(_B * _L, _H * _D)
    w3q = Wq_w.reshape(_C, _K, _ICH)
    w3k = Wk_w.reshape(_C, _K, _ICH)
    bq = jnp.transpose(Wq_b.reshape(_C, _K), (1, 0))
    bk = jnp.transpose(Wk_b.reshape(_C, _K), (1, 0))
    cntt = jnp.sum(
        (jnp.arange(_C)[:, None, None] == index_sample[None, :, :]),
        axis=2).astype(jnp.float32)

    qw, kw = pl.pallas_call(
        _proj_kernel,
        grid=(_NKB,),
        in_specs=[
            pl.BlockSpec((_B, _TL, _H, _D), lambda kb: (0, kb, 0, 0)),
            pl.BlockSpec((_B, _TL, _H, _D), lambda kb: (0, kb, 0, 0)),
            pl.BlockSpec((_C, _K, _KB), lambda kb: (0, 0, kb)),
            pl.BlockSpec((_C, _K, _KB), lambda kb: (0, 0, kb)),
            pl.BlockSpec((_K, _C), lambda kb: (0, 0)),
            pl.BlockSpec((_K, _C), lambda kb: (0, 0)),
        ],
        out_specs=[
            pl.BlockSpec((_BH, _K, _C), lambda kb: (0, 0, 0)),
            pl.BlockSpec((_BH, _K, _C), lambda kb: (0, 0, 0)),
        ],
        out_shape=[
            jax.ShapeDtypeStruct((_BH, _K, _C), jnp.float32),
            jax.ShapeDtypeStruct((_BH, _K, _C), jnp.float32),
        ],
        compiler_params=pltpu.CompilerParams(
            dimension_semantics=("arbitrary",)),
    )(queries, keys, w3q, w3k, bq, bk)

    ctx = pl.pallas_call(
        _attn_kernel,
        grid=(_BH // _G,),
        in_specs=[
            pl.BlockSpec((_G, _K, _C), lambda i: (i, 0, 0)),
            pl.BlockSpec((_G, _K, _C), lambda i: (i, 0, 0)),
            pl.BlockSpec((_G // _H * _L, _H * _D), lambda i: (i, 0)),
            pl.BlockSpec((_C, _C), lambda i: (0, 0)),
        ],
        out_specs=pl.BlockSpec((_G, _L, _D), lambda i: (i, 0, 0)),
        out_shape=jax.ShapeDtypeStruct((_BH, _L, _D), jnp.float32),
        compiler_params=pltpu.CompilerParams(
            dimension_semantics=("parallel",)),
    )(qw, kw, v2, cntt)

    return ctx.reshape(_B, _H, _L, _D)
